# Initial kernel scaffold; baseline (speedup 1.0000x reference)
import jax
import jax.numpy as jnp
from jax.experimental import pallas as pl


def kernel(ui_row, ui_col, ui_val, uu_row, uu_col, uu_val, ic_row, ic_col, ic_val, user_emb, item_emb, cat_emb):
    raise NotImplementedError("write your pallas kernel here")


# speedup vs baseline: 1.4241x; 1.4241x over previous
import functools

import jax
import jax.numpy as jnp
from jax import lax
from jax.experimental import pallas as pl
from jax.experimental.pallas import tpu as pltpu
from jax.experimental.pallas import tpu_sc as plsc

L = 16
NS = 16
NC = 2
B = 128
EB = NS * B
DIM = 128


def _spmm_body(rows_hbm, cols_hbm, vals_hbm, h_hbm, bounds_hbm, out_hbm,
               bvec_v, dst_v, idx_v, val_v, lidx_v, gbuf_v, zbuf_v,
               acc_sh, sem, *, n_tiles, T, SH):
    cid = lax.axis_index("c")
    sid = lax.axis_index("s")
    lane = lax.iota(jnp.int32, L)

    for r in range(L):
        for c in range(DIM // L):
            zbuf_v[r, pl.ds(c * L, L)] = jnp.zeros((L,), jnp.float32)

    pltpu.sync_copy(bounds_hbm, bvec_v)
    bv = bvec_v[...]

    for t in range(n_tiles):
        @pl.when(cid == (t % NC))
        def _tile():
            base = t * T
            zrows = SH // NS
            for j in range(zrows // L):
                pltpu.sync_copy(
                    zbuf_v, acc_sh.at[pl.ds(sid * zrows + j * L, L), :])
            plsc.subcore_barrier()

            lo = bv[2 * t]
            hi = bv[2 * t + 1]
            n = hi - lo
            chunk = lax.shift_left(
                lax.shift_right_logical(n + (EB - 1), 11), 7)
            slo = jnp.minimum(lo + sid * chunk, hi)
            shi = jnp.minimum(slo + chunk, hi)
            nb = lax.shift_right_logical(shi - slo, 7)

            def batch(b, carry):
                e = pl.multiple_of(slo + b * B, B)
                pltpu.sync_copy(rows_hbm.at[pl.ds(e, B)], dst_v)
                pltpu.sync_copy(cols_hbm.at[pl.ds(e, B)], idx_v)
                pltpu.sync_copy(vals_hbm.at[pl.ds(e, B)], val_v)
                pltpu.async_copy(h_hbm.at[idx_v], gbuf_v, sem).wait()

                for k in range(B // L):
                    d = dst_v[pl.ds(k * L, L)]
                    loc = d - base
                    ok = (loc >= 0) & (loc < T)
                    lidx_v[pl.ds(k * L, L)] = jnp.where(ok, loc, T)

                def scale(k, c2):
                    v = val_v[pl.ds(k * L, L)]
                    for j in range(L):
                        sv = jnp.zeros((L,), jnp.float32) + v[j]
                        row = k * L + j
                        for c in range(DIM // L):
                            gbuf_v[row, pl.ds(c * L, L)] = (
                                gbuf_v[row, pl.ds(c * L, L)] * sv)
                    return c2
                lax.fori_loop(0, B // L, scale, 0)

                pltpu.sync_copy(gbuf_v, acc_sh.at[lidx_v], add=True)
                return carry
            lax.fori_loop(0, nb, batch, 0)
            plsc.subcore_barrier()

            rps = T // NS
            for j in range(rps // L):
                off = sid * rps + j * L
                pltpu.sync_copy(acc_sh.at[pl.ds(off, L), :],
                                out_hbm.at[pl.ds(base + off, L), :])
            plsc.subcore_barrier()


@functools.lru_cache(maxsize=None)
def _make_spmm(e_pad, n_tiles, T, SH):
    mesh = plsc.VectorSubcoreMesh(core_axis_name="c", subcore_axis_name="s")
    return pl.kernel(
        functools.partial(_spmm_body, n_tiles=n_tiles, T=T, SH=SH),
        out_type=jax.ShapeDtypeStruct((n_tiles * T, DIM), jnp.float32),
        mesh=mesh,
        scratch_types=[
            pltpu.VMEM((L,), jnp.int32),
            pltpu.VMEM((B,), jnp.int32),
            pltpu.VMEM((B,), jnp.int32),
            pltpu.VMEM((B,), jnp.float32),
            pltpu.VMEM((B,), jnp.int32),
            pltpu.VMEM((B, DIM), jnp.float32),
            pltpu.VMEM((L, DIM), jnp.float32),
            pltpu.VMEM_SHARED((SH, DIM), jnp.float32),
            pltpu.SemaphoreType.DMA,
        ],
        name=f"spmm_e{e_pad}_t{n_tiles}x{T}",
    )


def _pad_edges(rows, cols, vals, big):
    e = rows.shape[0]
    e_pad = -(-e // EB) * EB
    pad = e_pad - e
    rows = jnp.concatenate(
        [rows.astype(jnp.int32), jnp.full((pad,), big, jnp.int32)])
    cols = jnp.concatenate([cols.astype(jnp.int32),
                            jnp.zeros((pad,), jnp.int32)])
    vals = jnp.concatenate([vals, jnp.zeros((pad,), jnp.float32)])
    return rows, cols, vals, e_pad


def _bounds_sorted(rows_p, e_pad, n_tiles, T):
    edges = jnp.arange(n_tiles + 1, dtype=jnp.int32) * T
    cuts = jnp.searchsorted(rows_p, edges).astype(jnp.int32)
    lo = (cuts[:-1] // B) * B
    hi = jnp.minimum(-(-cuts[1:] // B) * B, e_pad)
    b = jnp.stack([lo, hi], axis=1).reshape(-1)
    return jnp.concatenate(
        [b, jnp.zeros((L - 2 * n_tiles,), jnp.int32)])


def _bounds_full(e_pad, n_tiles):
    b = jnp.stack([jnp.zeros((n_tiles,), jnp.int32),
                   jnp.full((n_tiles,), e_pad, jnp.int32)], axis=1).reshape(-1)
    return jnp.concatenate([b, jnp.zeros((L - 2 * n_tiles,), jnp.int32)])


def _spmm(rows_p, cols_p, vals_p, h, bounds, e_pad, n_dst, n_tiles, T, SH):
    fn = _make_spmm(e_pad, n_tiles, T, SH)
    out = fn(rows_p, cols_p, vals_p, h, bounds)
    return out[:n_dst]


def kernel(ui_row, ui_col, ui_val, uu_row, uu_col, uu_val,
           ic_row, ic_col, ic_val, user_emb, item_emb, cat_emb):
    n_users = user_emb.shape[0]
    n_items = item_emb.shape[0]
    n_cats = cat_emb.shape[0]

    T_BIG = 12544
    NT_BIG = -(-max(n_users, n_items) // T_BIG)
    T_CAT = 512
    SH_BIG = T_BIG + 256
    SH_CAT = T_CAT + 256

    big = NT_BIG * T_BIG
    ui_r, ui_c, ui_v, e_ui = _pad_edges(ui_row, ui_col, ui_val, big)
    uu_r, uu_c, uu_v, e_uu = _pad_edges(uu_row, uu_col, uu_val, big)
    ic_r, ic_c, ic_v, e_ic = _pad_edges(ic_row, ic_col, ic_val, big)

    b_ui_fwd = _bounds_sorted(ui_r, e_ui, NT_BIG, T_BIG)
    b_ui_t = _bounds_full(e_ui, NT_BIG)
    b_uu = _bounds_sorted(uu_r, e_uu, NT_BIG, T_BIG)
    b_ic_fwd = _bounds_sorted(ic_r, e_ic, NT_BIG, T_BIG)
    b_ic_t = _bounds_full(e_ic, 1)

    h_u, h_i, h_c = user_emb, item_emb, cat_emb
    sum_u, sum_i = h_u, h_i
    for _ in range(2):
        m_u_i = _spmm(ui_r, ui_c, ui_v, h_i, b_ui_fwd,
                      e_ui, n_users, NT_BIG, T_BIG, SH_BIG)
        m_i_u = _spmm(ui_c, ui_r, ui_v, h_u, b_ui_t,
                      e_ui, n_items, NT_BIG, T_BIG, SH_BIG)
        m_u_u = _spmm(uu_r, uu_c, uu_v, h_u, b_uu,
                      e_uu, n_users, NT_BIG, T_BIG, SH_BIG)
        m_i_c = _spmm(ic_r, ic_c, ic_v, h_c, b_ic_fwd,
                      e_ic, n_items, NT_BIG, T_BIG, SH_BIG)
        m_c_i = _spmm(ic_c, ic_r, ic_v, h_i, b_ic_t,
                      e_ic, n_cats, 1, T_CAT, SH_CAT)
        h_u = m_u_i + m_u_u
        h_i = m_i_u + m_i_c
        h_c = m_c_i
        sum_u = sum_u + h_u
        sum_i = sum_i + h_i

    out_u = sum_u * jnp.float32(1.0 / 3.0)
    out_i = sum_i * jnp.float32(1.0 / 3.0)
    return (out_u, out_i, h_c)


# speedup vs baseline: 2.9700x; 2.0856x over previous
import functools

import jax
import jax.numpy as jnp
from jax import lax
from jax.experimental import pallas as pl
from jax.experimental.pallas import tpu as pltpu
from jax.experimental.pallas import tpu_sc as plsc

L = 16
NS = 16
NC = 2
B = 128
EB = NS * B
DIM = 128


def _spmm_body(rows_hbm, cols_hbm, vals_hbm, h_hbm, bounds_hbm, out_hbm,
               bvec_v, dst_v, idx_v, val_v, lidx_v, gbuf_v, zbuf_v,
               acc_sh, sem, *, n_tiles, T, SH):
    cid = lax.axis_index("c")
    sid = lax.axis_index("s")
    lane = lax.iota(jnp.int32, L)

    for r in range(L):
        for c in range(DIM // L):
            zbuf_v[r, pl.ds(c * L, L)] = jnp.zeros((L,), jnp.float32)

    pltpu.sync_copy(bounds_hbm, bvec_v)
    bv = bvec_v[...]

    for t in range(n_tiles):
        @pl.when(cid == (t % NC))
        def _tile():
            base = t * T
            zrows = SH // NS
            for j in range(zrows // L):
                pltpu.sync_copy(
                    zbuf_v, acc_sh.at[pl.ds(sid * zrows + j * L, L), :])
            plsc.subcore_barrier()

            lo = bv[2 * t]
            hi = bv[2 * t + 1]
            n = hi - lo
            chunk = lax.shift_left(
                lax.shift_right_logical(n + (EB - 1), 11), 7)
            slo = jnp.minimum(lo + sid * chunk, hi)
            shi = jnp.minimum(slo + chunk, hi)
            nb = lax.shift_right_logical(shi - slo, 7)

            def batch(b, carry):
                e = pl.multiple_of(slo + b * B, B)
                pltpu.sync_copy(rows_hbm.at[pl.ds(e, B)], dst_v)
                pltpu.sync_copy(cols_hbm.at[pl.ds(e, B)], idx_v)
                pltpu.sync_copy(vals_hbm.at[pl.ds(e, B)], val_v)
                pltpu.async_copy(h_hbm.at[idx_v], gbuf_v, sem).wait()

                for k in range(B // L):
                    d = dst_v[pl.ds(k * L, L)]
                    loc = d - base
                    ok = (loc >= 0) & (loc < T)
                    lidx_v[pl.ds(k * L, L)] = jnp.where(ok, loc, T)

                def scale(k, c2):
                    v = val_v[pl.ds(k * L, L)]
                    for j in range(L):
                        sv = jnp.zeros((L,), jnp.float32) + v[j]
                        row = k * L + j
                        for c in range(DIM // L):
                            gbuf_v[row, pl.ds(c * L, L)] = (
                                gbuf_v[row, pl.ds(c * L, L)] * sv)
                    return c2
                lax.fori_loop(0, B // L, scale, 0)

                pltpu.sync_copy(gbuf_v, acc_sh.at[lidx_v], add=True)
                return carry
            lax.fori_loop(0, nb, batch, 0)
            plsc.subcore_barrier()

            rps = T // NS
            for j in range(rps // L):
                off = sid * rps + j * L
                pltpu.sync_copy(acc_sh.at[pl.ds(off, L), :],
                                out_hbm.at[pl.ds(base + off, L), :])
            plsc.subcore_barrier()


@functools.lru_cache(maxsize=None)
def _make_spmm(e_pad, n_tiles, T, SH):
    mesh = plsc.VectorSubcoreMesh(core_axis_name="c", subcore_axis_name="s")
    return pl.kernel(
        functools.partial(_spmm_body, n_tiles=n_tiles, T=T, SH=SH),
        out_type=jax.ShapeDtypeStruct((n_tiles * T, DIM), jnp.float32),
        mesh=mesh,
        scratch_types=[
            pltpu.VMEM((L,), jnp.int32),
            pltpu.VMEM((B,), jnp.int32),
            pltpu.VMEM((B,), jnp.int32),
            pltpu.VMEM((B,), jnp.float32),
            pltpu.VMEM((B,), jnp.int32),
            pltpu.VMEM((B, DIM), jnp.float32),
            pltpu.VMEM((L, DIM), jnp.float32),
            pltpu.VMEM_SHARED((SH, DIM), jnp.float32),
            pltpu.SemaphoreType.DMA,
        ],
        name=f"spmm_e{e_pad}_t{n_tiles}x{T}",
    )


def _pad_edges(rows, cols, vals, big):
    e = rows.shape[0]
    e_pad = -(-e // EB) * EB
    pad = e_pad - e
    rows = jnp.concatenate(
        [rows.astype(jnp.int32), jnp.full((pad,), big, jnp.int32)])
    cols = jnp.concatenate([cols.astype(jnp.int32),
                            jnp.zeros((pad,), jnp.int32)])
    vals = jnp.concatenate([vals, jnp.zeros((pad,), jnp.float32)])
    return rows, cols, vals, e_pad


def _bounds_sorted(rows_p, e_pad, n_tiles, T):
    edges = jnp.arange(n_tiles + 1, dtype=jnp.int32) * T
    cuts = jnp.searchsorted(rows_p, edges).astype(jnp.int32)
    lo = (cuts[:-1] // B) * B
    hi = jnp.minimum(-(-cuts[1:] // B) * B, e_pad)
    b = jnp.stack([lo, hi], axis=1).reshape(-1)
    return jnp.concatenate(
        [b, jnp.zeros((L - 2 * n_tiles,), jnp.int32)])


def _bounds_full(e_pad, n_tiles):
    b = jnp.stack([jnp.zeros((n_tiles,), jnp.int32),
                   jnp.full((n_tiles,), e_pad, jnp.int32)], axis=1).reshape(-1)
    return jnp.concatenate([b, jnp.zeros((L - 2 * n_tiles,), jnp.int32)])


def _spmm(rows_p, cols_p, vals_p, h, bounds, e_pad, n_dst, n_tiles, T, SH):
    fn = _make_spmm(e_pad, n_tiles, T, SH)
    out = fn(rows_p, cols_p, vals_p, h, bounds)
    return out[:n_dst]


def kernel(ui_row, ui_col, ui_val, uu_row, uu_col, uu_val,
           ic_row, ic_col, ic_val, user_emb, item_emb, cat_emb):
    n_users = user_emb.shape[0]
    n_items = item_emb.shape[0]
    n_cats = cat_emb.shape[0]

    T_BIG = 12544
    NT_BIG = -(-max(n_users, n_items) // T_BIG)
    T_CAT = 512
    SH_BIG = T_BIG + 256
    SH_CAT = T_CAT + 256

    big = NT_BIG * T_BIG
    uit_row, uit_col, uit_val = lax.sort(
        (ui_col.astype(jnp.int32), ui_row.astype(jnp.int32), ui_val),
        num_keys=1)

    ui_r, ui_c, ui_v, e_ui = _pad_edges(ui_row, ui_col, ui_val, big)
    ut_r, ut_c, ut_v, _ = _pad_edges(uit_row, uit_col, uit_val, big)
    uu_r, uu_c, uu_v, e_uu = _pad_edges(uu_row, uu_col, uu_val, big)
    ic_r, ic_c, ic_v, e_ic = _pad_edges(ic_row, ic_col, ic_val, big)

    b_ui_fwd = _bounds_sorted(ui_r, e_ui, NT_BIG, T_BIG)
    b_ui_t = _bounds_sorted(ut_r, e_ui, NT_BIG, T_BIG)
    b_uu = _bounds_sorted(uu_r, e_uu, NT_BIG, T_BIG)
    b_ic_fwd = _bounds_sorted(ic_r, e_ic, NT_BIG, T_BIG)
    b_ic_t = _bounds_full(e_ic, 1)

    h_u, h_i, h_c = user_emb, item_emb, cat_emb
    sum_u, sum_i = h_u, h_i
    for _ in range(2):
        m_u_i = _spmm(ui_r, ui_c, ui_v, h_i, b_ui_fwd,
                      e_ui, n_users, NT_BIG, T_BIG, SH_BIG)
        m_i_u = _spmm(ut_r, ut_c, ut_v, h_u, b_ui_t,
                      e_ui, n_items, NT_BIG, T_BIG, SH_BIG)
        m_u_u = _spmm(uu_r, uu_c, uu_v, h_u, b_uu,
                      e_uu, n_users, NT_BIG, T_BIG, SH_BIG)
        m_i_c = _spmm(ic_r, ic_c, ic_v, h_c, b_ic_fwd,
                      e_ic, n_items, NT_BIG, T_BIG, SH_BIG)
        m_c_i = _spmm(ic_c, ic_r, ic_v, h_i, b_ic_t,
                      e_ic, n_cats, 1, T_CAT, SH_CAT)
        h_u = m_u_i + m_u_u
        h_i = m_i_u + m_i_c
        h_c = m_c_i
        sum_u = sum_u + h_u
        sum_i = sum_i + h_i

    out_u = sum_u * jnp.float32(1.0 / 3.0)
    out_i = sum_i * jnp.float32(1.0 / 3.0)
    return (out_u, out_i, h_c)


# speedup vs baseline: 3.8816x; 1.3069x over previous
import functools

import jax
import jax.numpy as jnp
from jax import lax
from jax.experimental import pallas as pl
from jax.experimental.pallas import tpu as pltpu
from jax.experimental.pallas import tpu_sc as plsc

L = 16
NS = 16
NC = 2
B = 64
B_LOG = 6
EB = NS * B
EB_LOG = 10
DIM = 128


CB = 8
CB_LOG = 3
CE = CB * B


def _spmm_body(rows_hbm, cols_hbm, vals_hbm, h_hbm, bounds_hbm, out_hbm,
               bvec_v, dst1k, idx1k, val1k, lidx_s, gbuf_s, zbuf_v,
               acc_sh, sem_g, sem_s, *, n_tiles, T, SH):
    cid = lax.axis_index("c")
    sid = lax.axis_index("s")

    for r in range(L):
        for c in range(DIM // L):
            zbuf_v[r, pl.ds(c * L, L)] = jnp.zeros((L,), jnp.float32)

    pltpu.sync_copy(bounds_hbm, bvec_v)
    bv = bvec_v[...]

    def drain_one_scatter():
        pltpu.make_async_copy(
            h_hbm.at[pl.ds(0, B), :], gbuf_s[0], sem_s).wait()

    def scale_batch(gbuf, j):
        def scale(k, c2):
            v = val1k[pl.ds(j * B + k * L, L)]
            for jj in range(L):
                sv = jnp.zeros((L,), jnp.float32) + v[jj]
                row = k * L + jj
                for c in range(DIM // L):
                    gbuf[row, pl.ds(c * L, L)] = (
                        gbuf[row, pl.ds(c * L, L)] * sv)
            return c2
        lax.fori_loop(0, B // L, scale, 0)

    lane = lax.iota(jnp.int32, L)

    def lidx_batch(lidx, j, base, ebase, shi):
        def body(k, c2):
            d = dst1k[pl.ds(j * B + k * L, L)]
            eidx = (ebase + (j * B + k * L)) + lane
            loc = d - base
            ok = (loc >= 0) & (loc < T) & (eidx < shi)
            lidx[pl.ds(k * L, L)] = jnp.where(ok, loc, T)
            return c2
        lax.fori_loop(0, B // L, body, 0)

    for t in range(n_tiles):
        @pl.when(cid == (t % NC))
        def _tile():
            base = t * T
            zrows = SH // NS
            for j in range(zrows // L):
                pltpu.sync_copy(
                    zbuf_v, acc_sh.at[pl.ds(sid * zrows + j * L, L), :])
            plsc.subcore_barrier()

            lo = bv[2 * t]
            hi = bv[2 * t + 1]
            n = hi - lo
            chunk = lax.shift_left(
                lax.shift_right_logical(n + (EB - 1), EB_LOG), B_LOG)
            slo = jnp.minimum(lo + sid * chunk, hi)
            shi = jnp.minimum(slo + chunk, hi)
            nb = lax.shift_right_logical(shi - slo, B_LOG)
            nsc = lax.shift_right_logical(nb + (CB - 1), CB_LOG)

            def super_chunk(s, carry):
                ec = pl.multiple_of(slo + s * CE, B)
                pltpu.sync_copy(rows_hbm.at[pl.ds(ec, CE)], dst1k)
                pltpu.sync_copy(cols_hbm.at[pl.ds(ec, CE)], idx1k)
                pltpu.sync_copy(vals_hbm.at[pl.ds(ec, CE)], val1k)

                pltpu.async_copy(
                    h_hbm.at[idx1k.at[pl.ds(0, B)]], gbuf_s[0], sem_g)
                for j in range(CB):
                    slot = j % 2
                    if j + 1 < CB:
                        if j >= 1:
                            drain_one_scatter()
                        pltpu.async_copy(
                            h_hbm.at[idx1k.at[pl.ds((j + 1) * B, B)]],
                            gbuf_s[(j + 1) % 2], sem_g)
                    pltpu.make_async_copy(
                        h_hbm.at[pl.ds(0, B), :], gbuf_s[slot],
                        sem_g).wait()
                    lidx_batch(lidx_s[slot], j, base, ec, shi)
                    scale_batch(gbuf_s[slot], j)
                    pltpu.async_copy(
                        gbuf_s[slot], acc_sh.at[lidx_s[slot]],
                        sem_s, add=True)
                drain_one_scatter()
                drain_one_scatter()
                return carry
            lax.fori_loop(0, nsc, super_chunk, 0)
            plsc.subcore_barrier()

            rps = T // NS
            for j in range(rps // L):
                off = sid * rps + j * L
                pltpu.sync_copy(acc_sh.at[pl.ds(off, L), :],
                                out_hbm.at[pl.ds(base + off, L), :])
            plsc.subcore_barrier()


@functools.lru_cache(maxsize=None)
def _make_spmm(e_pad, n_tiles, T, SH):
    mesh = plsc.VectorSubcoreMesh(core_axis_name="c", subcore_axis_name="s")
    return pl.kernel(
        functools.partial(_spmm_body, n_tiles=n_tiles, T=T, SH=SH),
        out_type=jax.ShapeDtypeStruct((n_tiles * T, DIM), jnp.float32),
        mesh=mesh,
        scratch_types=[
            pltpu.VMEM((L,), jnp.int32),
            pltpu.VMEM((CE,), jnp.int32),
            pltpu.VMEM((CE,), jnp.int32),
            pltpu.VMEM((CE,), jnp.float32),
            (pltpu.VMEM((B,), jnp.int32),
             pltpu.VMEM((B,), jnp.int32)),
            (pltpu.VMEM((B, DIM), jnp.float32),
             pltpu.VMEM((B, DIM), jnp.float32)),
            pltpu.VMEM((L, DIM), jnp.float32),
            pltpu.VMEM_SHARED((SH, DIM), jnp.float32),
            pltpu.SemaphoreType.DMA,
            pltpu.SemaphoreType.DMA,
        ],
        name=f"spmm_e{e_pad}_t{n_tiles}x{T}",
    )


def _pad_edges(rows, cols, vals, big):
    e = rows.shape[0]
    e_pad = -(-e // EB) * EB
    pad = e_pad + CE - e
    rows = jnp.concatenate(
        [rows.astype(jnp.int32), jnp.full((pad,), big, jnp.int32)])
    cols = jnp.concatenate([cols.astype(jnp.int32),
                            jnp.zeros((pad,), jnp.int32)])
    vals = jnp.concatenate([vals, jnp.zeros((pad,), jnp.float32)])
    return rows, cols, vals, e_pad


def _bounds_sorted(rows_p, e_pad, n_tiles, T):
    edges = jnp.arange(n_tiles + 1, dtype=jnp.int32) * T
    cuts = jnp.searchsorted(rows_p, edges).astype(jnp.int32)
    lo = (cuts[:-1] // B) * B
    hi = jnp.minimum(-(-cuts[1:] // B) * B, e_pad)
    b = jnp.stack([lo, hi], axis=1).reshape(-1)
    return jnp.concatenate(
        [b, jnp.zeros((L - 2 * n_tiles,), jnp.int32)])


def _bounds_full(e_pad, n_tiles):
    b = jnp.stack([jnp.zeros((n_tiles,), jnp.int32),
                   jnp.full((n_tiles,), e_pad, jnp.int32)], axis=1).reshape(-1)
    return jnp.concatenate([b, jnp.zeros((L - 2 * n_tiles,), jnp.int32)])


def _spmm(rows_p, cols_p, vals_p, h, bounds, e_pad, n_dst, n_tiles, T, SH):
    fn = _make_spmm(e_pad, n_tiles, T, SH)
    out = fn(rows_p, cols_p, vals_p, h, bounds)
    return out[:n_dst]


def kernel(ui_row, ui_col, ui_val, uu_row, uu_col, uu_val,
           ic_row, ic_col, ic_val, user_emb, item_emb, cat_emb):
    n_users = user_emb.shape[0]
    n_items = item_emb.shape[0]
    n_cats = cat_emb.shape[0]

    T_BIG = 13312
    NT_BIG = -(-max(n_users, n_items) // T_BIG)
    T_CAT = 512
    SH_BIG = T_BIG + 256
    SH_CAT = T_CAT + 256

    big = NT_BIG * T_BIG
    uit_row, uit_col, uit_val = lax.sort(
        (ui_col.astype(jnp.int32), ui_row.astype(jnp.int32), ui_val),
        num_keys=1)

    ui_r, ui_c, ui_v, e_ui = _pad_edges(ui_row, ui_col, ui_val, big)
    ut_r, ut_c, ut_v, _ = _pad_edges(uit_row, uit_col, uit_val, big)
    uu_r, uu_c, uu_v, e_uu = _pad_edges(uu_row, uu_col, uu_val, big)
    ic_r, ic_c, ic_v, e_ic = _pad_edges(ic_row, ic_col, ic_val, big)
    ct_r, ct_c, ct_v, _ = _pad_edges(ic_col, ic_row, ic_val, 0)

    b_ui_fwd = _bounds_sorted(ui_r, e_ui, NT_BIG, T_BIG)
    b_ui_t = _bounds_sorted(ut_r, e_ui, NT_BIG, T_BIG)
    b_uu = _bounds_sorted(uu_r, e_uu, NT_BIG, T_BIG)
    b_ic_fwd = _bounds_sorted(ic_r, e_ic, NT_BIG, T_BIG)
    b_ic_t = _bounds_full(e_ic, 1)

    h_u, h_i, h_c = user_emb, item_emb, cat_emb
    sum_u, sum_i = h_u, h_i
    for _ in range(2):
        m_u_i = _spmm(ui_r, ui_c, ui_v, h_i, b_ui_fwd,
                      e_ui, n_users, NT_BIG, T_BIG, SH_BIG)
        m_i_u = _spmm(ut_r, ut_c, ut_v, h_u, b_ui_t,
                      e_ui, n_items, NT_BIG, T_BIG, SH_BIG)
        m_u_u = _spmm(uu_r, uu_c, uu_v, h_u, b_uu,
                      e_uu, n_users, NT_BIG, T_BIG, SH_BIG)
        m_i_c = _spmm(ic_r, ic_c, ic_v, h_c, b_ic_fwd,
                      e_ic, n_items, NT_BIG, T_BIG, SH_BIG)
        m_c_i = _spmm(ct_r, ct_c, ct_v, h_i, b_ic_t,
                      e_ic, n_cats, 1, T_CAT, SH_CAT)
        h_u = m_u_i + m_u_u
        h_i = m_i_u + m_i_c
        h_c = m_c_i
        sum_u = sum_u + h_u
        sum_i = sum_i + h_i

    out_u = sum_u * jnp.float32(1.0 / 3.0)
    out_i = sum_i * jnp.float32(1.0 / 3.0)
    return (out_u, out_i, h_c)
